# baseline (speedup 1.0000x reference)
import functools

import jax
import jax.numpy as jnp
from jax import lax
from jax.experimental import pallas as pl
from jax.experimental.pallas import tpu as pltpu
from jax.experimental.pallas import tpu_sc as plsc

N_USERS = 1000000
N_FACTORS = 64
BATCH = 16384
NUM_WORKERS = 32
B_PER_W = BATCH // NUM_WORKERS
IDX_CHUNK = 128
N_CHUNKS = B_PER_W // IDX_CHUNK
N_GROUPS = B_PER_W // 16
W_ROWS = 8
G_ROWS = 16 * W_ROWS

N_WIN = N_USERS // 128
TAIL = N_USERS - N_WIN * 128
W_PER_TILE = -(-N_WIN // NUM_WORKERS)


def _tr_body(uft_hbm, tail_hbm, uscr_hbm, win0, win1, rowbuf0, rowbuf1,
             tailbuf, sem0, sem1, wsem0, wsem1):
    wid = lax.axis_index("s") * 2 + lax.axis_index("c")
    w0 = wid * W_PER_TILE
    teff = jnp.minimum(N_WIN - w0, W_PER_TILE)
    iota = lax.iota(jnp.int32, 16)
    wins = (win0, win1)
    rowbufs = (rowbuf0, rowbuf1)
    sems = (sem0, sem1)
    wsems = (wsem0, wsem1)

    def fetch(w, p):
        col = pl.multiple_of(w * 128, 128)
        pltpu.async_copy(uft_hbm.at[:, pl.ds(col, 128)], wins[p], sems[p])

    def wait_fetch(p):
        pltpu.make_async_copy(uft_hbm.at[:, pl.ds(0, 128)], wins[p],
                              sems[p]).wait()

    def transpose(p):
        win = wins[p]
        rowbuf = rowbufs[p]

        def row_body(l, carry):
            lvec = jnp.zeros((16,), jnp.int32) + l
            for k in range(4):
                u = plsc.load_gather(win, [k * 16 + iota, lvec])
                plsc.store_scatter(rowbuf, [lvec, k * 16 + iota], u)
            return carry

        lax.fori_loop(0, 128, row_body, 0)

    def store(w, p):
        row = pl.multiple_of(w * 128, 128)
        pltpu.async_copy(rowbufs[p], uscr_hbm.at[pl.ds(row, 128), :],
                         wsems[p])

    def wait_store(p):
        pltpu.make_async_copy(uscr_hbm.at[pl.ds(0, 128), :], rowbufs[p],
                              wsems[p]).wait()

    @pl.when(0 < teff)
    def _():
        fetch(w0, 0)

    @pl.when(1 < teff)
    def _():
        fetch(w0 + 1, 1)

    def win_body(s, carry):
        for p in range(2):
            t = s * 2 + p
            w = w0 + t

            @pl.when(t < teff)
            def _():
                wait_fetch(p)

                @pl.when(t >= 2)
                def _():
                    wait_store(p)

                transpose(p)

                @pl.when(t + 2 < teff)
                def _():
                    fetch(w + 2, p)

                store(w, p)

        return carry

    lax.fori_loop(0, (W_PER_TILE + 1) // 2, win_body, 0)

    @pl.when(teff >= 1)
    def _():
        wait_store(0)

    @pl.when(teff >= 2)
    def _():
        wait_store(1)

    @pl.when(wid == 0)
    def _():
        pltpu.sync_copy(tail_hbm, tailbuf)
        pltpu.sync_copy(tailbuf, uscr_hbm.at[pl.ds(N_WIN * 128, TAIL), :])


def _mf_body(uids_hbm, iids_hbm, uf_hbm, if_hbm, ub_hbm, ib_hbm, gb_hbm,
             out_hbm,
             idx_u, idx_i, urows0, urows1, irows0, irows1,
             bu0, bu1, bi0, bi1, gb_v, out_v, sem0, sem1):
    wid = lax.axis_index("s") * 2 + lax.axis_index("c")
    base = wid * B_PER_W

    for j in range(N_CHUNKS):
        src = pl.ds(base + j * IDX_CHUNK, IDX_CHUNK)
        dst = pl.ds(j * IDX_CHUNK, IDX_CHUNK)
        pltpu.sync_copy(uids_hbm.at[src], idx_u.at[dst])
        pltpu.sync_copy(iids_hbm.at[src], idx_i.at[dst])
    pltpu.sync_copy(gb_hbm, gb_v)
    gb = gb_v[...]

    def issue(g, urows, irows, bu, bi, sem):
        col0 = g * 16
        vu = idx_u[pl.ds(col0, 16)]
        vi = idx_i[pl.ds(col0, 16)]
        for l in range(16):
            ru = (vu[l] >> 3) << 3
            ri = (vi[l] >> 3) << 3
            ru = pl.multiple_of(ru, 8)
            ri = pl.multiple_of(ri, 8)
            dstw = pl.ds(l * W_ROWS, W_ROWS)
            pltpu.async_copy(uf_hbm.at[pl.ds(ru, W_ROWS), :],
                             urows.at[dstw, :], sem)
            pltpu.async_copy(if_hbm.at[pl.ds(ri, W_ROWS), :],
                             irows.at[dstw, :], sem)
            pltpu.async_copy(ub_hbm.at[pl.ds(ru, W_ROWS)], bu.at[dstw], sem)
            pltpu.async_copy(ib_hbm.at[pl.ds(ri, W_ROWS)], bi.at[dstw], sem)

    def drain(urows, irows, bu, bi, sem):
        pltpu.make_async_copy(uf_hbm.at[pl.ds(0, G_ROWS), :], urows, sem).wait()
        pltpu.make_async_copy(if_hbm.at[pl.ds(0, G_ROWS), :], irows, sem).wait()
        pltpu.make_async_copy(ub_hbm.at[pl.ds(0, G_ROWS)], bu, sem).wait()
        pltpu.make_async_copy(ib_hbm.at[pl.ds(0, G_ROWS)], bi, sem).wait()

    lane8 = lax.iota(jnp.int32, 16) * W_ROWS

    def compute(g, urows, irows, bu, bi):
        col0 = g * 16
        vu = idx_u[pl.ds(col0, 16)]
        vi = idx_i[pl.ds(col0, 16)]
        rows_u = lane8 + (vu & 7)
        rows_i = lane8 + (vi & 7)
        acc = (plsc.load_gather(bu, [rows_u])
               + plsc.load_gather(bi, [rows_i]) + gb)
        for d in range(N_FACTORS):
            drow = jnp.full((16,), d, jnp.int32)
            u = plsc.load_gather(urows, [rows_u, drow])
            v = plsc.load_gather(irows, [rows_i, drow])
            acc = acc + u * v
        out_v[pl.ds(col0, 16)] = acc

    issue(0, urows0, irows0, bu0, bi0, sem0)

    def pair_body(t, carry):
        g0 = t * 2
        g1 = g0 + 1
        issue(g1, urows1, irows1, bu1, bi1, sem1)
        drain(urows0, irows0, bu0, bi0, sem0)
        compute(g0, urows0, irows0, bu0, bi0)

        @pl.when(t < (N_GROUPS // 2 - 1))
        def _():
            issue(g0 + 2, urows0, irows0, bu0, bi0, sem0)

        drain(urows1, irows1, bu1, bi1, sem1)
        compute(g1, urows1, irows1, bu1, bi1)
        return carry

    lax.fori_loop(0, N_GROUPS // 2, pair_body, 0)

    pltpu.sync_copy(out_v, out_hbm.at[pl.ds(base, B_PER_W)])


_transpose_kernel = functools.partial(
    pl.kernel,
    mesh=plsc.VectorSubcoreMesh(core_axis_name="c", subcore_axis_name="s"),
    out_type=jax.ShapeDtypeStruct((N_USERS, N_FACTORS), jnp.float32),
    compiler_params=pltpu.CompilerParams(needs_layout_passes=False,
                                         use_tc_tiling_on_sc=True),
    scratch_types=[
        pltpu.VMEM((N_FACTORS, 128), jnp.float32),
        pltpu.VMEM((N_FACTORS, 128), jnp.float32),
        pltpu.VMEM((128, N_FACTORS), jnp.float32),
        pltpu.VMEM((128, N_FACTORS), jnp.float32),
        pltpu.VMEM((TAIL, N_FACTORS), jnp.float32),
        pltpu.SemaphoreType.DMA,
        pltpu.SemaphoreType.DMA,
        pltpu.SemaphoreType.DMA,
        pltpu.SemaphoreType.DMA,
    ],
)(_tr_body)


_mf_kernel = functools.partial(
    pl.kernel,
    mesh=plsc.VectorSubcoreMesh(core_axis_name="c", subcore_axis_name="s"),
    out_type=jax.ShapeDtypeStruct((BATCH,), jnp.float32),
    compiler_params=pltpu.CompilerParams(needs_layout_passes=False,
                                         use_tc_tiling_on_sc=True),
    scratch_types=[
        pltpu.VMEM((B_PER_W,), jnp.int32),
        pltpu.VMEM((B_PER_W,), jnp.int32),
        pltpu.VMEM((G_ROWS, N_FACTORS), jnp.float32),
        pltpu.VMEM((G_ROWS, N_FACTORS), jnp.float32),
        pltpu.VMEM((G_ROWS, N_FACTORS), jnp.float32),
        pltpu.VMEM((G_ROWS, N_FACTORS), jnp.float32),
        pltpu.VMEM((G_ROWS,), jnp.float32),
        pltpu.VMEM((G_ROWS,), jnp.float32),
        pltpu.VMEM((G_ROWS,), jnp.float32),
        pltpu.VMEM((G_ROWS,), jnp.float32),
        pltpu.VMEM((16,), jnp.float32),
        pltpu.VMEM((B_PER_W,), jnp.float32),
        pltpu.SemaphoreType.DMA,
        pltpu.SemaphoreType.DMA,
    ],
)(_mf_body)


@jax.jit
def kernel(user_ids, item_ids, user_factors, item_factors, user_bias,
           item_bias, global_bias):
    gb16 = jnp.broadcast_to(global_bias.reshape(()), (16,))
    tail_u = user_factors[N_WIN * 128:, :]
    u_scr = _transpose_kernel(user_factors.T, tail_u)
    return _mf_kernel(user_ids.astype(jnp.int32), item_ids.astype(jnp.int32),
                      u_scr, item_factors,
                      user_bias.reshape(-1), item_bias.reshape(-1), gb16)


# speedup vs baseline: 4.4043x; 4.4043x over previous
import functools

import jax
import jax.numpy as jnp
from jax import lax
from jax.experimental import pallas as pl
from jax.experimental.pallas import tpu as pltpu
from jax.experimental.pallas import tpu_sc as plsc

N_FACTORS = 64
BATCH = 16384
NUM_WORKERS = 32
B_PER_W = BATCH // NUM_WORKERS
IDX_CHUNK = 128
N_CHUNKS = B_PER_W // IDX_CHUNK
GSZ = 4
N_GROUPS = B_PER_W // GSZ
UW_ROWS = GSZ * N_FACTORS
W_ROWS = 8
G_ROWS = GSZ * W_ROWS


def _mf_body(uids_hbm, iids_hbm, uft_hbm, if_hbm, ub_hbm, ib_hbm, gb_hbm,
             out_hbm,
             idx_u, idx_i, uwin0, uwin1, irows0, irows1,
             bu0, bu1, bi0, bi1, gb_v, out_v, sem0, sem1):
    wid = lax.axis_index("s") * 2 + lax.axis_index("c")
    base = wid * B_PER_W

    for j in range(N_CHUNKS):
        src = pl.ds(base + j * IDX_CHUNK, IDX_CHUNK)
        dst = pl.ds(j * IDX_CHUNK, IDX_CHUNK)
        pltpu.sync_copy(uids_hbm.at[src], idx_u.at[dst])
        pltpu.sync_copy(iids_hbm.at[src], idx_i.at[dst])
    pltpu.sync_copy(gb_hbm, gb_v)
    gb = gb_v[...]
    iota = lax.iota(jnp.int32, 16)

    def group_ids(g):
        al = pl.multiple_of((g >> 2) * 16, 16)
        lane0 = lax.rem(g, GSZ) * GSZ
        sel = lane0 + (iota & 3)
        vua = idx_u[pl.ds(al, 16)].at[sel].get(mode="promise_in_bounds")
        via = idx_i[pl.ds(al, 16)].at[sel].get(mode="promise_in_bounds")
        return vua, via

    def issue(g, uwin, irows, bu, bi, sem):
        vu, vi = group_ids(g)
        for l in range(GSZ):
            cu = pl.multiple_of((vu[l] >> 7) * 128, 128)
            ri = pl.multiple_of((vi[l] >> 3) << 3, 8)
            ru = pl.multiple_of((vu[l] >> 3) << 3, 8)
            pltpu.async_copy(uft_hbm.at[:, pl.ds(cu, 128)],
                             uwin.at[pl.ds(l * N_FACTORS, N_FACTORS), :], sem)
            pltpu.async_copy(if_hbm.at[pl.ds(ri, W_ROWS), :],
                             irows.at[pl.ds(l * W_ROWS, W_ROWS), :], sem)
            pltpu.async_copy(ub_hbm.at[pl.ds(ru, W_ROWS)],
                             bu.at[pl.ds(l * W_ROWS, W_ROWS)], sem)
            pltpu.async_copy(ib_hbm.at[pl.ds(ri, W_ROWS)],
                             bi.at[pl.ds(l * W_ROWS, W_ROWS)], sem)

    def drain(uwin, irows, bu, bi, sem):
        pltpu.make_async_copy(uft_hbm.at[:, pl.ds(0, 128)],
                              uwin.at[pl.ds(0, N_FACTORS), :], sem).wait()
        pltpu.make_async_copy(uft_hbm.at[:, pl.ds(0, 128)],
                              uwin.at[pl.ds(N_FACTORS, N_FACTORS), :],
                              sem).wait()
        pltpu.make_async_copy(uft_hbm.at[:, pl.ds(0, 128)],
                              uwin.at[pl.ds(2 * N_FACTORS, N_FACTORS), :],
                              sem).wait()
        pltpu.make_async_copy(uft_hbm.at[:, pl.ds(0, 128)],
                              uwin.at[pl.ds(3 * N_FACTORS, N_FACTORS), :],
                              sem).wait()
        pltpu.make_async_copy(if_hbm.at[pl.ds(0, G_ROWS), :], irows,
                              sem).wait()
        pltpu.make_async_copy(ub_hbm.at[pl.ds(0, G_ROWS)], bu, sem).wait()
        pltpu.make_async_copy(ib_hbm.at[pl.ds(0, G_ROWS)], bi, sem).wait()

    def compute(g, uwin, irows, bu, bi, acc):
        vu, vi = group_ids(g)
        lane0 = lax.rem(g, GSZ) * GSZ
        for l in range(GSZ):
            ucol = jnp.zeros((16,), jnp.int32) + (vu[l] & 127)
            irow = jnp.zeros((16,), jnp.int32) + (l * W_ROWS + (vi[l] & 7))
            dot = jnp.zeros((16,), jnp.float32)
            for k in range(4):
                ch = k * 16 + iota
                u = plsc.load_gather(uwin, [l * N_FACTORS + ch, ucol])
                v = plsc.load_gather(irows, [irow, ch])
                dot = dot + u * v
            s = jnp.sum(dot)
            ubv = plsc.load_gather(bu, [jnp.zeros((16,), jnp.int32)
                                        + (l * W_ROWS + (vu[l] & 7))])
            ibv = plsc.load_gather(bi, [irow])
            s = s + ubv[0] + ibv[0]
            acc = jnp.where(iota == lane0 + l, s, acc)
        return acc

    issue(0, uwin0, irows0, bu0, bi0, sem0)

    def pair_body(t, acc):
        g0 = t * 2
        g1 = g0 + 1
        issue(g1, uwin1, irows1, bu1, bi1, sem1)
        drain(uwin0, irows0, bu0, bi0, sem0)
        acc = compute(g0, uwin0, irows0, bu0, bi0, acc)

        @pl.when(t < (N_GROUPS // 2 - 1))
        def _():
            issue(g0 + 2, uwin0, irows0, bu0, bi0, sem0)

        drain(uwin1, irows1, bu1, bi1, sem1)
        acc = compute(g1, uwin1, irows1, bu1, bi1, acc)

        @pl.when(lax.rem(t, 2) == 1)
        def _():
            out_v[pl.ds((g0 - 2) * GSZ, 16)] = acc + gb

        return acc

    lax.fori_loop(0, N_GROUPS // 2, pair_body, jnp.zeros((16,), jnp.float32))

    pltpu.sync_copy(out_v, out_hbm.at[pl.ds(base, B_PER_W)])


_mf_kernel = functools.partial(
    pl.kernel,
    mesh=plsc.VectorSubcoreMesh(core_axis_name="c", subcore_axis_name="s"),
    out_type=jax.ShapeDtypeStruct((BATCH,), jnp.float32),
    compiler_params=pltpu.CompilerParams(needs_layout_passes=False,
                                         use_tc_tiling_on_sc=True),
    scratch_types=[
        pltpu.VMEM((B_PER_W + 16,), jnp.int32),
        pltpu.VMEM((B_PER_W + 16,), jnp.int32),
        pltpu.VMEM((UW_ROWS, 128), jnp.float32),
        pltpu.VMEM((UW_ROWS, 128), jnp.float32),
        pltpu.VMEM((G_ROWS, N_FACTORS), jnp.float32),
        pltpu.VMEM((G_ROWS, N_FACTORS), jnp.float32),
        pltpu.VMEM((G_ROWS,), jnp.float32),
        pltpu.VMEM((G_ROWS,), jnp.float32),
        pltpu.VMEM((G_ROWS,), jnp.float32),
        pltpu.VMEM((G_ROWS,), jnp.float32),
        pltpu.VMEM((16,), jnp.float32),
        pltpu.VMEM((B_PER_W,), jnp.float32),
        pltpu.SemaphoreType.DMA,
        pltpu.SemaphoreType.DMA,
    ],
)(_mf_body)


@jax.jit
def kernel(user_ids, item_ids, user_factors, item_factors, user_bias,
           item_bias, global_bias):
    gb16 = jnp.broadcast_to(global_bias.reshape(()), (16,))
    return _mf_kernel(user_ids.astype(jnp.int32), item_ids.astype(jnp.int32),
                      user_factors.T, item_factors,
                      user_bias.reshape(-1), item_bias.reshape(-1), gb16)


# speedup vs baseline: 4.9691x; 1.1282x over previous
import functools

import jax
import jax.numpy as jnp
from jax import lax
from jax.experimental import pallas as pl
from jax.experimental.pallas import tpu as pltpu
from jax.experimental.pallas import tpu_sc as plsc

N_FACTORS = 64
BATCH = 16384
NUM_WORKERS = 32
B_PER_W = BATCH // NUM_WORKERS
IDX_CHUNK = 128
N_CHUNKS = B_PER_W // IDX_CHUNK
GSZ = 4
N_SUPER = B_PER_W // 16
UW_ROWS = GSZ * N_FACTORS
W_ROWS = 8
G_ROWS = 16 * W_ROWS


def _ug_body(uids_hbm, uft_hbm, urows_hbm,
             idx_u, uwin0, uwin1, stage0, stage1, sem0, sem1, wsem0, wsem1):
    wid = lax.axis_index("s") * 2 + lax.axis_index("c")
    base = wid * B_PER_W

    for j in range(N_CHUNKS):
        pltpu.sync_copy(uids_hbm.at[pl.ds(base + j * IDX_CHUNK, IDX_CHUNK)],
                        idx_u.at[pl.ds(j * IDX_CHUNK, IDX_CHUNK)])

    iota = lax.iota(jnp.int32, 16)
    uwins = (uwin0, uwin1)
    sems = (sem0, sem1)

    def issue(vu16, p, pbuf):
        for l in range(GSZ):
            cu = pl.multiple_of((vu16[p * GSZ + l] >> 7) * 128, 128)
            pltpu.async_copy(uft_hbm.at[:, pl.ds(cu, 128)],
                             uwins[pbuf].at[pl.ds(l * N_FACTORS, N_FACTORS), :],
                             sems[pbuf])

    def drain(pbuf):
        for l in range(GSZ):
            pltpu.make_async_copy(
                uft_hbm.at[:, pl.ds(0, 128)],
                uwins[pbuf].at[pl.ds(l * N_FACTORS, N_FACTORS), :],
                sems[pbuf]).wait()

    def extract(vu16, p, pbuf, stage, slot0):
        for l in range(GSZ):
            uid = vu16[p * GSZ + l]
            ucol = jnp.zeros((16,), jnp.int32) + (uid & 127)
            for k in range(4):
                u = plsc.load_gather(uwins[pbuf],
                                     [l * N_FACTORS + k * 16 + iota, ucol])
                stage[slot0 + l, pl.ds(k * 16, 16)] = u

    def ids_at(s):
        al = pl.multiple_of(s * 16, 16)
        return idx_u[pl.ds(al, 16)]

    issue(ids_at(0), 0, 0)

    def super_body(s, carry):
        vu = ids_at(s)
        vun = ids_at(jnp.minimum(s + 1, N_SUPER - 1))

        @pl.when(s >= 1)
        def _():
            pltpu.make_async_copy(urows_hbm.at[pl.ds(0, 8), :], stage0,
                                  wsem0).wait()
            pltpu.make_async_copy(urows_hbm.at[pl.ds(0, 8), :], stage1,
                                  wsem1).wait()

        issue(vu, 1, 1)
        drain(0)
        extract(vu, 0, 0, stage0, 0)
        issue(vu, 2, 0)
        drain(1)
        extract(vu, 1, 1, stage0, 4)
        pltpu.async_copy(stage0, urows_hbm.at[pl.ds(base + s * 16, 8), :],
                         wsem0)
        issue(vu, 3, 1)
        drain(0)
        extract(vu, 2, 0, stage1, 0)

        @pl.when(s < N_SUPER - 1)
        def _():
            issue(vun, 0, 0)

        drain(1)
        extract(vu, 3, 1, stage1, 4)
        pltpu.async_copy(stage1, urows_hbm.at[pl.ds(base + s * 16 + 8, 8), :],
                         wsem1)
        return carry

    lax.fori_loop(0, N_SUPER, super_body, 0)

    pltpu.make_async_copy(urows_hbm.at[pl.ds(0, 8), :], stage0, wsem0).wait()
    pltpu.make_async_copy(urows_hbm.at[pl.ds(0, 8), :], stage1, wsem1).wait()


def _mf2_body(uids_hbm, iids_hbm, ur_hbm, if_hbm, ub_hbm, ib_hbm, gb_hbm,
              out_hbm,
              idx_u, idx_i, ubuf0, ubuf1, irows0, irows1,
              bu0, bu1, bi0, bi1, gb_v, out_v, sem0, sem1):
    wid = lax.axis_index("s") * 2 + lax.axis_index("c")
    base = wid * B_PER_W

    for j in range(N_CHUNKS):
        src = pl.ds(base + j * IDX_CHUNK, IDX_CHUNK)
        dst = pl.ds(j * IDX_CHUNK, IDX_CHUNK)
        pltpu.sync_copy(uids_hbm.at[src], idx_u.at[dst])
        pltpu.sync_copy(iids_hbm.at[src], idx_i.at[dst])
    pltpu.sync_copy(gb_hbm, gb_v)
    gb = gb_v[...]

    def issue(g, ubuf, irows, bu, bi, sem):
        col0 = g * 16
        vu = idx_u[pl.ds(col0, 16)]
        vi = idx_i[pl.ds(col0, 16)]
        row = pl.multiple_of(base + col0, 16)
        pltpu.async_copy(ur_hbm.at[pl.ds(row, 16), :], ubuf, sem)
        for l in range(16):
            ru = pl.multiple_of((vu[l] >> 3) << 3, 8)
            ri = pl.multiple_of((vi[l] >> 3) << 3, 8)
            dstw = pl.ds(l * W_ROWS, W_ROWS)
            pltpu.async_copy(if_hbm.at[pl.ds(ri, W_ROWS), :],
                             irows.at[dstw, :], sem)
            pltpu.async_copy(ub_hbm.at[pl.ds(ru, W_ROWS)], bu.at[dstw], sem)
            pltpu.async_copy(ib_hbm.at[pl.ds(ri, W_ROWS)], bi.at[dstw], sem)

    def drain(ubuf, irows, bu, bi, sem):
        pltpu.make_async_copy(ur_hbm.at[pl.ds(0, 16), :], ubuf, sem).wait()
        pltpu.make_async_copy(if_hbm.at[pl.ds(0, G_ROWS), :], irows,
                              sem).wait()
        pltpu.make_async_copy(ub_hbm.at[pl.ds(0, G_ROWS)], bu, sem).wait()
        pltpu.make_async_copy(ib_hbm.at[pl.ds(0, G_ROWS)], bi, sem).wait()

    iota = lax.iota(jnp.int32, 16)
    lane8 = iota * W_ROWS

    def compute(g, ubuf, irows, bu, bi):
        col0 = g * 16
        vu = idx_u[pl.ds(col0, 16)]
        vi = idx_i[pl.ds(col0, 16)]
        rows_i = lane8 + (vi & 7)
        acc = (plsc.load_gather(bu, [lane8 + (vu & 7)])
               + plsc.load_gather(bi, [rows_i]) + gb)
        for d in range(N_FACTORS):
            drow = jnp.full((16,), d, jnp.int32)
            u = plsc.load_gather(ubuf, [iota, drow])
            v = plsc.load_gather(irows, [rows_i, drow])
            acc = acc + u * v
        out_v[pl.ds(col0, 16)] = acc

    issue(0, ubuf0, irows0, bu0, bi0, sem0)

    def pair_body(t, carry):
        g0 = t * 2
        g1 = g0 + 1
        issue(g1, ubuf1, irows1, bu1, bi1, sem1)
        drain(ubuf0, irows0, bu0, bi0, sem0)
        compute(g0, ubuf0, irows0, bu0, bi0)

        @pl.when(t < (N_SUPER // 2 - 1))
        def _():
            issue(g0 + 2, ubuf0, irows0, bu0, bi0, sem0)

        drain(ubuf1, irows1, bu1, bi1, sem1)
        compute(g1, ubuf1, irows1, bu1, bi1)
        return carry

    lax.fori_loop(0, N_SUPER // 2, pair_body, 0)

    pltpu.sync_copy(out_v, out_hbm.at[pl.ds(base, B_PER_W)])


_ug_kernel = functools.partial(
    pl.kernel,
    mesh=plsc.VectorSubcoreMesh(core_axis_name="c", subcore_axis_name="s"),
    out_type=jax.ShapeDtypeStruct((BATCH, N_FACTORS), jnp.float32),
    compiler_params=pltpu.CompilerParams(needs_layout_passes=False,
                                         use_tc_tiling_on_sc=True),
    scratch_types=[
        pltpu.VMEM((B_PER_W,), jnp.int32),
        pltpu.VMEM((UW_ROWS, 128), jnp.float32),
        pltpu.VMEM((UW_ROWS, 128), jnp.float32),
        pltpu.VMEM((8, N_FACTORS), jnp.float32),
        pltpu.VMEM((8, N_FACTORS), jnp.float32),
        pltpu.SemaphoreType.DMA,
        pltpu.SemaphoreType.DMA,
        pltpu.SemaphoreType.DMA,
        pltpu.SemaphoreType.DMA,
    ],
)(_ug_body)


_mf2_kernel = functools.partial(
    pl.kernel,
    mesh=plsc.VectorSubcoreMesh(core_axis_name="c", subcore_axis_name="s"),
    out_type=jax.ShapeDtypeStruct((BATCH,), jnp.float32),
    compiler_params=pltpu.CompilerParams(needs_layout_passes=False,
                                         use_tc_tiling_on_sc=True),
    scratch_types=[
        pltpu.VMEM((B_PER_W,), jnp.int32),
        pltpu.VMEM((B_PER_W,), jnp.int32),
        pltpu.VMEM((16, N_FACTORS), jnp.float32),
        pltpu.VMEM((16, N_FACTORS), jnp.float32),
        pltpu.VMEM((G_ROWS, N_FACTORS), jnp.float32),
        pltpu.VMEM((G_ROWS, N_FACTORS), jnp.float32),
        pltpu.VMEM((G_ROWS,), jnp.float32),
        pltpu.VMEM((G_ROWS,), jnp.float32),
        pltpu.VMEM((G_ROWS,), jnp.float32),
        pltpu.VMEM((G_ROWS,), jnp.float32),
        pltpu.VMEM((16,), jnp.float32),
        pltpu.VMEM((B_PER_W,), jnp.float32),
        pltpu.SemaphoreType.DMA,
        pltpu.SemaphoreType.DMA,
    ],
)(_mf2_body)


@jax.jit
def kernel(user_ids, item_ids, user_factors, item_factors, user_bias,
           item_bias, global_bias):
    gb16 = jnp.broadcast_to(global_bias.reshape(()), (16,))
    uids = user_ids.astype(jnp.int32)
    iids = item_ids.astype(jnp.int32)
    u_rows = _ug_kernel(uids, user_factors.T)
    return _mf2_kernel(uids, iids, u_rows, item_factors,
                       user_bias.reshape(-1), item_bias.reshape(-1), gb16)


# speedup vs baseline: 5.3365x; 1.0739x over previous
import functools

import jax
import jax.numpy as jnp
from jax import lax
from jax.experimental import pallas as pl
from jax.experimental.pallas import tpu as pltpu
from jax.experimental.pallas import tpu_sc as plsc

N_FACTORS = 64
BATCH = 16384
NUM_WORKERS = 32
B_PER_W = BATCH // NUM_WORKERS
IDX_CHUNK = 128
N_CHUNKS = B_PER_W // IDX_CHUNK
UGSZ = 2
N_SUPER = B_PER_W // 16
UW_ROWS = UGSZ * N_FACTORS
W_ROWS = 8
G_ROWS = 16 * W_ROWS


def _ug_body(uids_hbm, uft_hbm, urows_hbm,
             idx_u, uwin0, uwin1, uwin2, uwin3, stage0, stage1,
             sem0, sem1, sem2, sem3, wsem0, wsem1):
    wid = lax.axis_index("s") * 2 + lax.axis_index("c")
    base = wid * B_PER_W

    for j in range(N_CHUNKS):
        pltpu.sync_copy(uids_hbm.at[pl.ds(base + j * IDX_CHUNK, IDX_CHUNK)],
                        idx_u.at[pl.ds(j * IDX_CHUNK, IDX_CHUNK)])

    iota = lax.iota(jnp.int32, 16)
    uwins = (uwin0, uwin1, uwin2, uwin3)
    sems = (sem0, sem1, sem2, sem3)

    def issue(vu16, q, b):
        for l in range(UGSZ):
            cu = pl.multiple_of((vu16[q * UGSZ + l] >> 7) * 128, 128)
            pltpu.async_copy(uft_hbm.at[:, pl.ds(cu, 128)],
                             uwins[b].at[pl.ds(l * N_FACTORS, N_FACTORS), :],
                             sems[b])

    def drain(b):
        for l in range(UGSZ):
            pltpu.make_async_copy(
                uft_hbm.at[:, pl.ds(0, 128)],
                uwins[b].at[pl.ds(l * N_FACTORS, N_FACTORS), :],
                sems[b]).wait()

    def extract(vu16, q, b, stage, slot0):
        for l in range(UGSZ):
            uid = vu16[q * UGSZ + l]
            ucol = jnp.zeros((16,), jnp.int32) + (uid & 127)
            for k in range(4):
                u = plsc.load_gather(uwins[b],
                                     [l * N_FACTORS + k * 16 + iota, ucol])
                stage[slot0 + l, pl.ds(k * 16, 16)] = u

    def ids_at(s):
        al = pl.multiple_of(s * 16, 16)
        return idx_u[pl.ds(al, 16)]

    for q in range(4):
        issue(ids_at(0), q, q)

    def super_body(s, carry):
        vu = ids_at(s)
        vun = ids_at(jnp.minimum(s + 1, N_SUPER - 1))

        @pl.when(s >= 1)
        def _():
            pltpu.make_async_copy(urows_hbm.at[pl.ds(0, 8), :], stage0,
                                  wsem0).wait()
            pltpu.make_async_copy(urows_hbm.at[pl.ds(0, 8), :], stage1,
                                  wsem1).wait()

        for q in range(8):
            b = q % 4
            stage, slot0 = (stage0, q * UGSZ) if q < 4 else \
                           (stage1, (q - 4) * UGSZ)
            drain(b)
            extract(vu, q, b, stage, slot0)
            if q < 4:
                issue(vu, q + 4, b)
            else:
                @pl.when(s < N_SUPER - 1)
                def _(q=q, b=b, vun=vun):
                    issue(vun, q - 4, b)
            if q == 3:
                pltpu.async_copy(stage0,
                                 urows_hbm.at[pl.ds(base + s * 16, 8), :],
                                 wsem0)
            if q == 7:
                pltpu.async_copy(stage1,
                                 urows_hbm.at[pl.ds(base + s * 16 + 8, 8), :],
                                 wsem1)
        return carry

    lax.fori_loop(0, N_SUPER, super_body, 0)

    pltpu.make_async_copy(urows_hbm.at[pl.ds(0, 8), :], stage0, wsem0).wait()
    pltpu.make_async_copy(urows_hbm.at[pl.ds(0, 8), :], stage1, wsem1).wait()


def _mf2_body(uids_hbm, iids_hbm, ur_hbm, if_hbm, ub_hbm, ib_hbm, gb_hbm,
              out_hbm,
              idx_u, idx_i, ubuf0, ubuf1, irows0, irows1,
              bu0, bu1, bi0, bi1, gb_v, out_v, sem0, sem1):
    wid = lax.axis_index("s") * 2 + lax.axis_index("c")
    base = wid * B_PER_W

    for j in range(N_CHUNKS):
        src = pl.ds(base + j * IDX_CHUNK, IDX_CHUNK)
        dst = pl.ds(j * IDX_CHUNK, IDX_CHUNK)
        pltpu.sync_copy(uids_hbm.at[src], idx_u.at[dst])
        pltpu.sync_copy(iids_hbm.at[src], idx_i.at[dst])
    pltpu.sync_copy(gb_hbm, gb_v)
    gb = gb_v[...]

    def issue(g, ubuf, irows, bu, bi, sem):
        col0 = g * 16
        vu = idx_u[pl.ds(col0, 16)]
        vi = idx_i[pl.ds(col0, 16)]
        row = pl.multiple_of(base + col0, 16)
        pltpu.async_copy(ur_hbm.at[pl.ds(row, 16), :], ubuf, sem)
        for l in range(16):
            ru = pl.multiple_of((vu[l] >> 3) << 3, 8)
            ri = pl.multiple_of((vi[l] >> 3) << 3, 8)
            dstw = pl.ds(l * W_ROWS, W_ROWS)
            pltpu.async_copy(if_hbm.at[pl.ds(ri, W_ROWS), :],
                             irows.at[dstw, :], sem)
            pltpu.async_copy(ub_hbm.at[pl.ds(ru, W_ROWS)], bu.at[dstw], sem)
            pltpu.async_copy(ib_hbm.at[pl.ds(ri, W_ROWS)], bi.at[dstw], sem)

    def drain(ubuf, irows, bu, bi, sem):
        pltpu.make_async_copy(ur_hbm.at[pl.ds(0, 16), :], ubuf, sem).wait()
        pltpu.make_async_copy(if_hbm.at[pl.ds(0, G_ROWS), :], irows,
                              sem).wait()
        pltpu.make_async_copy(ub_hbm.at[pl.ds(0, G_ROWS)], bu, sem).wait()
        pltpu.make_async_copy(ib_hbm.at[pl.ds(0, G_ROWS)], bi, sem).wait()

    iota = lax.iota(jnp.int32, 16)
    lane8 = iota * W_ROWS

    def compute(g, ubuf, irows, bu, bi):
        col0 = g * 16
        vu = idx_u[pl.ds(col0, 16)]
        vi = idx_i[pl.ds(col0, 16)]
        rows_i = lane8 + (vi & 7)
        acc = (plsc.load_gather(bu, [lane8 + (vu & 7)])
               + plsc.load_gather(bi, [rows_i]) + gb)
        for d in range(N_FACTORS):
            drow = jnp.full((16,), d, jnp.int32)
            u = plsc.load_gather(ubuf, [iota, drow])
            v = plsc.load_gather(irows, [rows_i, drow])
            acc = acc + u * v
        out_v[pl.ds(col0, 16)] = acc

    issue(0, ubuf0, irows0, bu0, bi0, sem0)

    def pair_body(t, carry):
        g0 = t * 2
        g1 = g0 + 1
        issue(g1, ubuf1, irows1, bu1, bi1, sem1)
        drain(ubuf0, irows0, bu0, bi0, sem0)
        compute(g0, ubuf0, irows0, bu0, bi0)

        @pl.when(t < (N_SUPER // 2 - 1))
        def _():
            issue(g0 + 2, ubuf0, irows0, bu0, bi0, sem0)

        drain(ubuf1, irows1, bu1, bi1, sem1)
        compute(g1, ubuf1, irows1, bu1, bi1)
        return carry

    lax.fori_loop(0, N_SUPER // 2, pair_body, 0)

    pltpu.sync_copy(out_v, out_hbm.at[pl.ds(base, B_PER_W)])


_ug_kernel = functools.partial(
    pl.kernel,
    mesh=plsc.VectorSubcoreMesh(core_axis_name="c", subcore_axis_name="s"),
    out_type=jax.ShapeDtypeStruct((BATCH, N_FACTORS), jnp.float32),
    compiler_params=pltpu.CompilerParams(needs_layout_passes=False,
                                         use_tc_tiling_on_sc=True),
    scratch_types=[
        pltpu.VMEM((B_PER_W,), jnp.int32),
        pltpu.VMEM((UW_ROWS, 128), jnp.float32),
        pltpu.VMEM((UW_ROWS, 128), jnp.float32),
        pltpu.VMEM((UW_ROWS, 128), jnp.float32),
        pltpu.VMEM((UW_ROWS, 128), jnp.float32),
        pltpu.VMEM((8, N_FACTORS), jnp.float32),
        pltpu.VMEM((8, N_FACTORS), jnp.float32),
        pltpu.SemaphoreType.DMA,
        pltpu.SemaphoreType.DMA,
        pltpu.SemaphoreType.DMA,
        pltpu.SemaphoreType.DMA,
        pltpu.SemaphoreType.DMA,
        pltpu.SemaphoreType.DMA,
    ],
)(_ug_body)


_mf2_kernel = functools.partial(
    pl.kernel,
    mesh=plsc.VectorSubcoreMesh(core_axis_name="c", subcore_axis_name="s"),
    out_type=jax.ShapeDtypeStruct((BATCH,), jnp.float32),
    compiler_params=pltpu.CompilerParams(needs_layout_passes=False,
                                         use_tc_tiling_on_sc=True),
    scratch_types=[
        pltpu.VMEM((B_PER_W,), jnp.int32),
        pltpu.VMEM((B_PER_W,), jnp.int32),
        pltpu.VMEM((16, N_FACTORS), jnp.float32),
        pltpu.VMEM((16, N_FACTORS), jnp.float32),
        pltpu.VMEM((G_ROWS, N_FACTORS), jnp.float32),
        pltpu.VMEM((G_ROWS, N_FACTORS), jnp.float32),
        pltpu.VMEM((G_ROWS,), jnp.float32),
        pltpu.VMEM((G_ROWS,), jnp.float32),
        pltpu.VMEM((G_ROWS,), jnp.float32),
        pltpu.VMEM((G_ROWS,), jnp.float32),
        pltpu.VMEM((16,), jnp.float32),
        pltpu.VMEM((B_PER_W,), jnp.float32),
        pltpu.SemaphoreType.DMA,
        pltpu.SemaphoreType.DMA,
    ],
)(_mf2_body)


@jax.jit
def kernel(user_ids, item_ids, user_factors, item_factors, user_bias,
           item_bias, global_bias):
    gb16 = jnp.broadcast_to(global_bias.reshape(()), (16,))
    uids = user_ids.astype(jnp.int32)
    iids = item_ids.astype(jnp.int32)
    u_rows = _ug_kernel(uids, user_factors.T)
    return _mf2_kernel(uids, iids, u_rows, item_factors,
                       user_bias.reshape(-1), item_bias.reshape(-1), gb16)


# speedup vs baseline: 5.6013x; 1.0496x over previous
import functools

import jax
import jax.numpy as jnp
from jax import lax
from jax.experimental import pallas as pl
from jax.experimental.pallas import tpu as pltpu
from jax.experimental.pallas import tpu_sc as plsc

N_FACTORS = 64
BATCH = 16384
NUM_WORKERS = 32
B_PER_W = BATCH // NUM_WORKERS
IDX_CHUNK = 128
N_CHUNKS = B_PER_W // IDX_CHUNK
UGSZ = 1
N_SUPER = B_PER_W // 16
UW_ROWS = UGSZ * N_FACTORS
W_ROWS = 8
G_ROWS = 16 * W_ROWS


def _ug_body(uids_hbm, uft_hbm, urows_hbm,
             idx_u, uwin0, uwin1, uwin2, uwin3, uwin4, uwin5, uwin6, uwin7,
             stage0, stage1, sem0, sem1, sem2, sem3, sem4, sem5, sem6, sem7,
             wsem0, wsem1):
    wid = lax.axis_index("s") * 2 + lax.axis_index("c")
    base = wid * B_PER_W

    for j in range(N_CHUNKS):
        pltpu.sync_copy(uids_hbm.at[pl.ds(base + j * IDX_CHUNK, IDX_CHUNK)],
                        idx_u.at[pl.ds(j * IDX_CHUNK, IDX_CHUNK)])

    iota = lax.iota(jnp.int32, 16)
    uwins = (uwin0, uwin1, uwin2, uwin3, uwin4, uwin5, uwin6, uwin7)
    sems = (sem0, sem1, sem2, sem3, sem4, sem5, sem6, sem7)

    def issue(vu16, q, b):
        for l in range(UGSZ):
            cu = pl.multiple_of((vu16[q * UGSZ + l] >> 7) * 128, 128)
            pltpu.async_copy(uft_hbm.at[:, pl.ds(cu, 128)],
                             uwins[b].at[pl.ds(l * N_FACTORS, N_FACTORS), :],
                             sems[b])

    def drain(b):
        for l in range(UGSZ):
            pltpu.make_async_copy(
                uft_hbm.at[:, pl.ds(0, 128)],
                uwins[b].at[pl.ds(l * N_FACTORS, N_FACTORS), :],
                sems[b]).wait()

    def extract(vu16, q, b, stage, slot0):
        for l in range(UGSZ):
            uid = vu16[q * UGSZ + l]
            ucol = jnp.zeros((16,), jnp.int32) + (uid & 127)
            for k in range(4):
                u = plsc.load_gather(uwins[b],
                                     [l * N_FACTORS + k * 16 + iota, ucol])
                stage[slot0 + l, pl.ds(k * 16, 16)] = u

    def ids_at(s):
        al = pl.multiple_of(s * 16, 16)
        return idx_u[pl.ds(al, 16)]

    for q in range(8):
        issue(ids_at(0), q, q)

    def super_body(s, carry):
        vu = ids_at(s)
        vun = ids_at(jnp.minimum(s + 1, N_SUPER - 1))

        @pl.when(s >= 1)
        def _():
            pltpu.make_async_copy(urows_hbm.at[pl.ds(0, 8), :], stage0,
                                  wsem0).wait()
            pltpu.make_async_copy(urows_hbm.at[pl.ds(0, 8), :], stage1,
                                  wsem1).wait()

        for q in range(16):
            b = q % 8
            stage, slot0 = (stage0, q * UGSZ) if q < 8 else \
                           (stage1, (q - 8) * UGSZ)
            drain(b)
            extract(vu, q, b, stage, slot0)
            if q < 8:
                issue(vu, q + 8, b)
            else:
                @pl.when(s < N_SUPER - 1)
                def _(q=q, b=b, vun=vun):
                    issue(vun, q - 8, b)
            if q == 7:
                pltpu.async_copy(stage0,
                                 urows_hbm.at[pl.ds(base + s * 16, 8), :],
                                 wsem0)
            if q == 15:
                pltpu.async_copy(stage1,
                                 urows_hbm.at[pl.ds(base + s * 16 + 8, 8), :],
                                 wsem1)
        return carry

    lax.fori_loop(0, N_SUPER, super_body, 0)

    pltpu.make_async_copy(urows_hbm.at[pl.ds(0, 8), :], stage0, wsem0).wait()
    pltpu.make_async_copy(urows_hbm.at[pl.ds(0, 8), :], stage1, wsem1).wait()


def _mf2_body(uids_hbm, iids_hbm, ur_hbm, if_hbm, ub_hbm, ib_hbm, gb_hbm,
              out_hbm,
              idx_u, idx_i, ubuf0, ubuf1, irows0, irows1,
              bu0, bu1, bi0, bi1, gb_v, out_v, sem0, sem1):
    wid = lax.axis_index("s") * 2 + lax.axis_index("c")
    base = wid * B_PER_W

    for j in range(N_CHUNKS):
        src = pl.ds(base + j * IDX_CHUNK, IDX_CHUNK)
        dst = pl.ds(j * IDX_CHUNK, IDX_CHUNK)
        pltpu.sync_copy(uids_hbm.at[src], idx_u.at[dst])
        pltpu.sync_copy(iids_hbm.at[src], idx_i.at[dst])
    pltpu.sync_copy(gb_hbm, gb_v)
    gb = gb_v[...]

    def issue(g, ubuf, irows, bu, bi, sem):
        col0 = g * 16
        vu = idx_u[pl.ds(col0, 16)]
        vi = idx_i[pl.ds(col0, 16)]
        row = pl.multiple_of(base + col0, 16)
        pltpu.async_copy(ur_hbm.at[pl.ds(row, 16), :], ubuf, sem)
        for l in range(16):
            ru = pl.multiple_of((vu[l] >> 3) << 3, 8)
            ri = pl.multiple_of((vi[l] >> 3) << 3, 8)
            dstw = pl.ds(l * W_ROWS, W_ROWS)
            pltpu.async_copy(if_hbm.at[pl.ds(ri, W_ROWS), :],
                             irows.at[dstw, :], sem)
            pltpu.async_copy(ub_hbm.at[pl.ds(ru, W_ROWS)], bu.at[dstw], sem)
            pltpu.async_copy(ib_hbm.at[pl.ds(ri, W_ROWS)], bi.at[dstw], sem)

    def drain(ubuf, irows, bu, bi, sem):
        pltpu.make_async_copy(ur_hbm.at[pl.ds(0, 16), :], ubuf, sem).wait()
        pltpu.make_async_copy(if_hbm.at[pl.ds(0, G_ROWS), :], irows,
                              sem).wait()
        pltpu.make_async_copy(ub_hbm.at[pl.ds(0, G_ROWS)], bu, sem).wait()
        pltpu.make_async_copy(ib_hbm.at[pl.ds(0, G_ROWS)], bi, sem).wait()

    iota = lax.iota(jnp.int32, 16)
    lane8 = iota * W_ROWS

    def compute(g, ubuf, irows, bu, bi):
        col0 = g * 16
        vu = idx_u[pl.ds(col0, 16)]
        vi = idx_i[pl.ds(col0, 16)]
        rows_i = lane8 + (vi & 7)
        acc = (plsc.load_gather(bu, [lane8 + (vu & 7)])
               + plsc.load_gather(bi, [rows_i]) + gb)
        for d in range(N_FACTORS):
            drow = jnp.full((16,), d, jnp.int32)
            u = plsc.load_gather(ubuf, [iota, drow])
            v = plsc.load_gather(irows, [rows_i, drow])
            acc = acc + u * v
        out_v[pl.ds(col0, 16)] = acc

    issue(0, ubuf0, irows0, bu0, bi0, sem0)

    def pair_body(t, carry):
        g0 = t * 2
        g1 = g0 + 1
        issue(g1, ubuf1, irows1, bu1, bi1, sem1)
        drain(ubuf0, irows0, bu0, bi0, sem0)
        compute(g0, ubuf0, irows0, bu0, bi0)

        @pl.when(t < (N_SUPER // 2 - 1))
        def _():
            issue(g0 + 2, ubuf0, irows0, bu0, bi0, sem0)

        drain(ubuf1, irows1, bu1, bi1, sem1)
        compute(g1, ubuf1, irows1, bu1, bi1)
        return carry

    lax.fori_loop(0, N_SUPER // 2, pair_body, 0)

    pltpu.sync_copy(out_v, out_hbm.at[pl.ds(base, B_PER_W)])


_ug_kernel = functools.partial(
    pl.kernel,
    mesh=plsc.VectorSubcoreMesh(core_axis_name="c", subcore_axis_name="s"),
    out_type=jax.ShapeDtypeStruct((BATCH, N_FACTORS), jnp.float32),
    compiler_params=pltpu.CompilerParams(needs_layout_passes=False,
                                         use_tc_tiling_on_sc=True),
    scratch_types=[
        pltpu.VMEM((B_PER_W,), jnp.int32),
        pltpu.VMEM((UW_ROWS, 128), jnp.float32),
        pltpu.VMEM((UW_ROWS, 128), jnp.float32),
        pltpu.VMEM((UW_ROWS, 128), jnp.float32),
        pltpu.VMEM((UW_ROWS, 128), jnp.float32),
        pltpu.VMEM((UW_ROWS, 128), jnp.float32),
        pltpu.VMEM((UW_ROWS, 128), jnp.float32),
        pltpu.VMEM((UW_ROWS, 128), jnp.float32),
        pltpu.VMEM((UW_ROWS, 128), jnp.float32),
        pltpu.VMEM((8, N_FACTORS), jnp.float32),
        pltpu.VMEM((8, N_FACTORS), jnp.float32),
        pltpu.SemaphoreType.DMA,
        pltpu.SemaphoreType.DMA,
        pltpu.SemaphoreType.DMA,
        pltpu.SemaphoreType.DMA,
        pltpu.SemaphoreType.DMA,
        pltpu.SemaphoreType.DMA,
        pltpu.SemaphoreType.DMA,
        pltpu.SemaphoreType.DMA,
        pltpu.SemaphoreType.DMA,
        pltpu.SemaphoreType.DMA,
    ],
)(_ug_body)


_mf2_kernel = functools.partial(
    pl.kernel,
    mesh=plsc.VectorSubcoreMesh(core_axis_name="c", subcore_axis_name="s"),
    out_type=jax.ShapeDtypeStruct((BATCH,), jnp.float32),
    compiler_params=pltpu.CompilerParams(needs_layout_passes=False,
                                         use_tc_tiling_on_sc=True),
    scratch_types=[
        pltpu.VMEM((B_PER_W,), jnp.int32),
        pltpu.VMEM((B_PER_W,), jnp.int32),
        pltpu.VMEM((16, N_FACTORS), jnp.float32),
        pltpu.VMEM((16, N_FACTORS), jnp.float32),
        pltpu.VMEM((G_ROWS, N_FACTORS), jnp.float32),
        pltpu.VMEM((G_ROWS, N_FACTORS), jnp.float32),
        pltpu.VMEM((G_ROWS,), jnp.float32),
        pltpu.VMEM((G_ROWS,), jnp.float32),
        pltpu.VMEM((G_ROWS,), jnp.float32),
        pltpu.VMEM((G_ROWS,), jnp.float32),
        pltpu.VMEM((16,), jnp.float32),
        pltpu.VMEM((B_PER_W,), jnp.float32),
        pltpu.SemaphoreType.DMA,
        pltpu.SemaphoreType.DMA,
    ],
)(_mf2_body)


@jax.jit
def kernel(user_ids, item_ids, user_factors, item_factors, user_bias,
           item_bias, global_bias):
    gb16 = jnp.broadcast_to(global_bias.reshape(()), (16,))
    uids = user_ids.astype(jnp.int32)
    iids = item_ids.astype(jnp.int32)
    u_rows = _ug_kernel(uids, user_factors.T)
    return _mf2_kernel(uids, iids, u_rows, item_factors,
                       user_bias.reshape(-1), item_bias.reshape(-1), gb16)
